# Initial kernel scaffold; baseline (speedup 1.0000x reference)
import jax
import jax.numpy as jnp
from jax.experimental import pallas as pl


def kernel(dense_in, sparse_idx, W1, b1, W2, b2, W3, b3, W4, b4, table):
    raise NotImplementedError("write your pallas kernel here")


# speedup vs baseline: 1.1351x; 1.1351x over previous
import functools

import jax
import jax.numpy as jnp
from jax import lax
from jax.experimental import pallas as pl
from jax.experimental.pallas import tpu as pltpu
from jax.experimental.pallas import tpu_sc as plsc

B = 16384
NJ = 26
E = 64
NC, NS = 2, 16
NW = NC * NS
SPW = B // NW
CH_S = 16
CH_P = CH_S * NJ
NSTR = 4
STR_P = CH_P // NSTR
NCH = SPW // CH_S


def _sc_emb_dot(idx2d, table, w_emb):
    @functools.partial(
        pl.kernel,
        out_type=jax.ShapeDtypeStruct((B,), jnp.float32),
        mesh=plsc.VectorSubcoreMesh(core_axis_name="c", subcore_axis_name="s"),
        compiler_params=pltpu.CompilerParams(use_tc_tiling_on_sc=False),
        scratch_types=[
            pltpu.VMEM((NCH * NSTR, STR_P), jnp.int32),
            pltpu.VMEM((CH_P, E), jnp.float32),
            pltpu.VMEM((CH_P, E), jnp.float32),
            pltpu.VMEM((NJ, E), jnp.float32),
            pltpu.VMEM((SPW,), jnp.float32),
            pltpu.SemaphoreType.DMA,
            pltpu.SemaphoreType.DMA,
        ],
    )
    def k(idx_hbm, table_hbm, w_hbm, out_hbm,
          idx_v, rows0, rows1, w_v, out_v, sem0, sem1):
        wid = lax.axis_index("s") * NC + lax.axis_index("c")
        pltpu.sync_copy(idx_hbm.at[pl.ds(wid * NCH * NSTR, NCH * NSTR)], idx_v)
        pltpu.sync_copy(w_hbm, w_v)

        rows = (rows0, rows1)
        sems = (sem0, sem1)
        lane = lax.iota(jnp.int32, 16)

        def fire(c, par):
            for q in range(NSTR):
                pltpu.async_copy(
                    table_hbm.at[idx_v.at[c * NSTR + q]],
                    rows[par].at[pl.ds(q * STR_P, STR_P)], sems[par])

        def drain(c, par):
            for q in range(NSTR):
                pltpu.make_async_copy(
                    table_hbm.at[idx_v.at[c * NSTR + q]],
                    rows[par].at[pl.ds(q * STR_P, STR_P)], sems[par]).wait()

        fire(0, 0)

        def compute_chunk(c, rbuf):
            vec = jnp.zeros((16,), jnp.float32)
            for s in range(CH_S):
                zero = jnp.zeros((16,), jnp.float32)

                def jbody(j, accs):
                    r = s * NJ + j
                    return tuple(
                        accs[t]
                        + rbuf[r, pl.ds(t * 16, 16)] * w_v[j, pl.ds(t * 16, 16)]
                        for t in range(4)
                    )

                a = lax.fori_loop(0, NJ, jbody, (zero, zero, zero, zero))
                tot = a[0] + a[1] + a[2] + a[3]
                for sh in (8, 4, 2, 1):
                    perm = jnp.bitwise_xor(lane, sh)
                    tot = tot + tot.at[perm].get(mode="promise_in_bounds")
                vec = jnp.where(lane == s, tot, vec)
            out_v[pl.ds(c * CH_S, CH_S)] = vec

        def body2(cc, carry):
            for par in range(2):
                c = cc * 2 + par
                nxt = c + 1

                @pl.when(nxt < NCH)
                def _(nxt=nxt, par=par):
                    fire(nxt, (par + 1) % 2)

                drain(c, par)
                compute_chunk(c, rows[par])
            return carry

        lax.fori_loop(0, NCH // 2, body2, 0)
        pltpu.sync_copy(out_v, out_hbm.at[pl.ds(wid * SPW, SPW)])

    return k(idx2d, table, w_emb)


BLK = 2048


def _mlp_body(x_ref, w1_ref, b1_ref, w2_ref, b2_ref, w3_ref, b3_ref,
              w4t_ref, b4_ref, emb_ref, out_ref):
    hi = jax.lax.Precision.HIGHEST
    x = x_ref[...]
    h1 = jnp.maximum(jnp.dot(x, w1_ref[...], precision=hi) + b1_ref[...], 0.0)
    h2 = jnp.maximum(jnp.dot(h1, w2_ref[...], precision=hi) + b2_ref[...], 0.0)
    v = jnp.dot(w3_ref[...], w4t_ref[...], precision=hi)
    c0 = jnp.dot(b3_ref[...], w4t_ref[...], precision=hi)
    out_ref[...] = (jnp.dot(h2, v, precision=hi)
                    + c0 + b4_ref[...] + emb_ref[...])


def _mlp(xpad, W1p, b1, W2, b2, W3, b3, w4t, b4, emb):
    full = lambda shape: pl.BlockSpec(shape, lambda i: (0, 0))
    return pl.pallas_call(
        _mlp_body,
        grid=(B // BLK,),
        in_specs=[
            pl.BlockSpec((BLK, 16), lambda i: (i, 0)),
            full((16, 256)), full((1, 256)),
            full((256, 256)), full((1, 256)),
            full((256, 256)), full((1, 256)),
            full((256, 1)), full((1, 1)),
            pl.BlockSpec((BLK, 1), lambda i: (i, 0)),
        ],
        out_specs=pl.BlockSpec((BLK, 1), lambda i: (i, 0)),
        out_shape=jax.ShapeDtypeStruct((B, 1), jnp.float32),
    )(xpad, W1p, b1, W2, b2, W3, b3, w4t, b4, emb)


def kernel(dense_in, sparse_idx, W1, b1, W2, b2, W3, b3, W4, b4, table):
    idx2d = sparse_idx.astype(jnp.int32).reshape(NW * NCH * NSTR, STR_P)
    w_emb = W4[:NJ * E, 0].reshape(NJ, E)
    w4t = W4[NJ * E:]
    emb = _sc_emb_dot(idx2d, table, w_emb)
    xpad = jnp.pad(dense_in, ((0, 0), (0, 3)))
    W1p = jnp.pad(W1, ((0, 3), (0, 0)))
    return _mlp(xpad, W1p, b1.reshape(1, -1), W2, b2.reshape(1, -1),
                W3, b3.reshape(1, -1), w4t, b4.reshape(1, 1),
                emb.reshape(B, 1))
